# baseline (speedup 1.0000x reference)
import functools

import jax
import jax.numpy as jnp
import numpy as np
from jax.experimental import pallas as pl

N = 10000
E = 320000
D = 128
EMB = 128
HEADS = 32
NUM_ATOM = 128
MASK_RATIO = 0.3
REPLACE_RATIO = 0.1
NOISE_VAL = 0.1
ALPHA_L = 2.0

NUM_MASK = int(MASK_RATIO * N)
NUM_NOISE = int(REPLACE_RATIO * NUM_MASK)
NUM_TOKEN = int((1.0 - REPLACE_RATIO) * NUM_MASK)


def _mask_constants():
    key = jax.random.key(42)
    k1, k2, k3, k4 = jax.random.split(key, 4)
    perm = jax.random.permutation(k1, N)
    mask_nodes = perm[:NUM_MASK].astype(jnp.int32)
    perm_mask = jax.random.permutation(k2, NUM_MASK)
    token_nodes = mask_nodes[perm_mask[:NUM_TOKEN]]
    noise_nodes = mask_nodes[perm_mask[-NUM_NOISE:]]
    noise_chosen = jax.random.permutation(k3, N)[:NUM_NOISE].astype(jnp.int32)
    pos_noise = NOISE_VAL * jax.random.normal(k4, (NUM_MASK, 3), jnp.float32)

    token_flag = jnp.zeros((N,), jnp.bool_).at[token_nodes].set(True)
    src_idx = jnp.arange(N, dtype=jnp.int32).at[noise_nodes].set(noise_chosen)
    u_add = jnp.zeros((N, 3), jnp.float32).at[mask_nodes].set(pos_noise)
    mask_set = jnp.zeros((N,), jnp.bool_).at[mask_nodes].set(True)
    return token_flag, src_idx, u_add, mask_nodes, mask_set


def _layer_norm(x, g, b, eps=1e-5):
    m = jnp.mean(x, axis=-1, keepdims=True)
    v = jnp.var(x, axis=-1, keepdims=True)
    return (x - m) / jnp.sqrt(v + eps) * g + b


def _enc_body(outx_ref, agg_ref, ws_ref, wa_ref, h_ref):
    acc = jnp.dot(outx_ref[...], ws_ref[...],
                  preferred_element_type=jnp.float32)
    acc += jnp.dot(agg_ref[...], wa_ref[...],
                   preferred_element_type=jnp.float32)
    h_ref[...] = jax.nn.gelu(acc)


def _encoder_h(out_x, agg, W_self, W_agg):
    blk = 1000
    grid = N // blk
    return pl.pallas_call(
        _enc_body,
        grid=(grid,),
        in_specs=[
            pl.BlockSpec((blk, D), lambda i: (i, 0)),
            pl.BlockSpec((blk, D), lambda i: (i, 0)),
            pl.BlockSpec((D, EMB), lambda i: (0, 0)),
            pl.BlockSpec((D, EMB), lambda i: (0, 0)),
        ],
        out_specs=pl.BlockSpec((blk, EMB), lambda i: (i, 0)),
        out_shape=jax.ShapeDtypeStruct((N, EMB), jnp.float32),
    )(out_x, agg, W_self, W_agg)


def kernel(x, edge_index, u, PE, edge_index_pe, enc_mask_token, W_self,
           W_agg, W_pe, mlm_dense_w, mlm_dense_b, mlm_ln_g, mlm_ln_b,
           mlm_weight, mlm_bias, dh_dense_w, dh_dense_b, dh_ln_g, dh_ln_b,
           dh_out_w, dh_out_b):
    token_flag, src_idx, u_add, mask_nodes, mask_set = _mask_constants()

    out_x = jnp.where(token_flag[:, None], enc_mask_token[0][None, :],
                      x[src_idx])
    u_masked = u + u_add

    pe_a, pe_b = edge_index_pe[0], edge_index_pe[1]
    ss = jnp.zeros((E,), jnp.float32)
    for c in range(3):
        uc = u_masked[:, c]
        dc = uc[pe_a] - uc[pe_b]
        ss = ss + dc * dc
    PE_noise = jnp.sqrt(ss + 1e-12)

    src, dst = edge_index[0], edge_index[1]
    agg = jax.ops.segment_sum(out_x[src], dst, num_segments=N)
    deg = jax.ops.segment_sum(jnp.ones((E,), jnp.float32), dst,
                              num_segments=N)
    agg = agg / jnp.maximum(deg, 1.0)[:, None]

    h = _encoder_h(out_x, agg, W_self, W_agg)

    hW = h @ W_pe
    pe = jnp.tanh(hW[edge_index_pe[0]] - hW[edge_index_pe[1]]
                  + (PE - PE_noise)[:, None])

    feats = h[mask_nodes]
    z = jax.nn.gelu(feats @ mlm_dense_w + mlm_dense_b)
    z = _layer_norm(z, mlm_ln_g, mlm_ln_b)
    pred_node = z @ mlm_weight.T + mlm_bias

    valid_e = edge_index_pe[0] != edge_index_pe[1]
    d = jax.nn.gelu(pe @ dh_dense_w + dh_dense_b)
    d = _layer_norm(d, dh_ln_g, dh_ln_b)
    d = d @ dh_out_w + dh_out_b

    target = x[mask_nodes]
    xn = pred_node / (jnp.linalg.norm(pred_node, axis=-1, keepdims=True)
                      + 1e-12)
    yn = target / (jnp.linalg.norm(target, axis=-1, keepdims=True) + 1e-12)
    atom_loss = jnp.mean((1.0 - jnp.sum(xn * yn, axis=-1)) ** ALPHA_L)

    sentinel = N * N
    e0 = jnp.concatenate([edge_index_pe[0], edge_index_pe[1]])
    e1 = jnp.concatenate([edge_index_pe[1], edge_index_pe[0]])
    valid2 = e0 != e1
    keys_ = jnp.where(valid2, e0 * N + e1, sentinel)
    M = keys_.shape[0]
    dflat = d[:, 0]
    d2 = jnp.concatenate([dflat, dflat])
    pe2 = jnp.concatenate([PE, PE])
    idx = jnp.arange(M, dtype=jnp.int32)
    sk, sidx = jax.lax.sort((keys_, idx), num_keys=1)
    ds = d2[sidx]
    ps = pe2[sidx]
    is_new = jnp.concatenate(
        [jnp.ones((1,), bool), sk[1:] != sk[:-1]])
    bnd = jnp.where(is_new, idx, M)
    suf = jnp.flip(jax.lax.cummin(jnp.flip(bnd)))
    s_next = jnp.concatenate([suf[1:], jnp.full((1,), M, jnp.int32)])
    e_last = s_next - 1
    dmean = jnp.mean(ds)
    pmean = jnp.mean(ps)
    Cd = jnp.cumsum(ds - dmean)
    Cp = jnp.cumsum(ps - pmean)
    sum_d = Cd[e_last] - (Cd - (ds - dmean))
    sum_p = Cp[e_last] - (Cp - (ps - pmean))
    cntf = (e_last - idx + 1).astype(jnp.float32)
    rec = sum_d / cntf + dmean
    tgt = sum_p / cntf + pmean
    i_p = jnp.clip(sk // N, 0, N - 1)
    maskf = mask_set.astype(jnp.float32)
    sel_w = (is_new & (sk != sentinel)).astype(jnp.float32) * maskf[i_p]
    dd = rec - tgt
    ad = jnp.abs(dd)
    row_loss = jnp.where(ad < 1.0, 0.5 * dd * dd, ad - 0.5)
    pe_loss = jnp.sum(row_loss * sel_w) / jnp.sum(sel_w)
    return atom_loss + pe_loss


# speedup vs baseline: 3.8539x; 3.8539x over previous
import functools

import jax
import jax.numpy as jnp
import numpy as np
from jax.experimental import pallas as pl

N = 10000
E = 320000
D = 128
EMB = 128
HEADS = 32
NUM_ATOM = 128
MASK_RATIO = 0.3
REPLACE_RATIO = 0.1
NOISE_VAL = 0.1
ALPHA_L = 2.0

NUM_MASK = int(MASK_RATIO * N)
NUM_NOISE = int(REPLACE_RATIO * NUM_MASK)
NUM_TOKEN = int((1.0 - REPLACE_RATIO) * NUM_MASK)


def _mask_constants():
    key = jax.random.key(42)
    k1, k2, k3, k4 = jax.random.split(key, 4)
    perm = jax.random.permutation(k1, N)
    mask_nodes = perm[:NUM_MASK].astype(jnp.int32)
    perm_mask = jax.random.permutation(k2, NUM_MASK)
    token_nodes = mask_nodes[perm_mask[:NUM_TOKEN]]
    noise_nodes = mask_nodes[perm_mask[-NUM_NOISE:]]
    noise_chosen = jax.random.permutation(k3, N)[:NUM_NOISE].astype(jnp.int32)
    pos_noise = NOISE_VAL * jax.random.normal(k4, (NUM_MASK, 3), jnp.float32)

    token_flag = jnp.zeros((N,), jnp.bool_).at[token_nodes].set(True)
    src_idx = jnp.arange(N, dtype=jnp.int32).at[noise_nodes].set(noise_chosen)
    u_add = jnp.zeros((N, 3), jnp.float32).at[mask_nodes].set(pos_noise)
    mask_set = jnp.zeros((N,), jnp.bool_).at[mask_nodes].set(True)
    return token_flag, src_idx, u_add, mask_nodes, mask_set


def _layer_norm(x, g, b, eps=1e-5):
    m = jnp.mean(x, axis=-1, keepdims=True)
    v = jnp.var(x, axis=-1, keepdims=True)
    return (x - m) / jnp.sqrt(v + eps) * g + b


def _enc_body(outx_ref, agg_ref, ws_ref, wa_ref, h_ref):
    acc = jnp.dot(outx_ref[...], ws_ref[...],
                  preferred_element_type=jnp.float32)
    acc += jnp.dot(agg_ref[...], wa_ref[...],
                   preferred_element_type=jnp.float32)
    h_ref[...] = jax.nn.gelu(acc)


def _encoder_h(out_x, agg, W_self, W_agg):
    blk = 1000
    grid = N // blk
    return pl.pallas_call(
        _enc_body,
        grid=(grid,),
        in_specs=[
            pl.BlockSpec((blk, D), lambda i: (i, 0)),
            pl.BlockSpec((blk, D), lambda i: (i, 0)),
            pl.BlockSpec((D, EMB), lambda i: (0, 0)),
            pl.BlockSpec((D, EMB), lambda i: (0, 0)),
        ],
        out_specs=pl.BlockSpec((blk, EMB), lambda i: (i, 0)),
        out_shape=jax.ShapeDtypeStruct((N, EMB), jnp.float32),
    )(out_x, agg, W_self, W_agg)


def kernel(x, edge_index, u, PE, edge_index_pe, enc_mask_token, W_self,
           W_agg, W_pe, mlm_dense_w, mlm_dense_b, mlm_ln_g, mlm_ln_b,
           mlm_weight, mlm_bias, dh_dense_w, dh_dense_b, dh_ln_g, dh_ln_b,
           dh_out_w, dh_out_b):
    token_flag, src_idx, u_add, mask_nodes, mask_set = _mask_constants()

    out_x = jnp.where(token_flag[:, None], enc_mask_token[0][None, :],
                      x[src_idx])
    u_masked = u + u_add

    pe_a, pe_b = edge_index_pe[0], edge_index_pe[1]
    maskf = mask_set.astype(jnp.float32)
    U16 = jnp.concatenate(
        [u_masked, maskf[:, None], jnp.zeros((N, 12), jnp.float32)], axis=1)
    Ua = U16[pe_a]
    Ub = U16[pe_b]
    du = Ua[:, :3] - Ub[:, :3]
    PE_noise = jnp.sqrt(jnp.sum(du * du, axis=-1) + 1e-12)
    mask_a = Ua[:, 3].astype(jnp.int32)
    mask_b = Ub[:, 3].astype(jnp.int32)

    src, dst = edge_index[0], edge_index[1]
    agg = jax.ops.segment_sum(out_x[src], dst, num_segments=N)
    deg = jax.ops.segment_sum(jnp.ones((E,), jnp.float32), dst,
                              num_segments=N)
    agg = agg / jnp.maximum(deg, 1.0)[:, None]

    h = _encoder_h(out_x, agg, W_self, W_agg)

    hW = h @ W_pe
    pe = jnp.tanh(hW[edge_index_pe[0]] - hW[edge_index_pe[1]]
                  + (PE - PE_noise)[:, None])

    feats = h[mask_nodes]
    z = jax.nn.gelu(feats @ mlm_dense_w + mlm_dense_b)
    z = _layer_norm(z, mlm_ln_g, mlm_ln_b)
    pred_node = z @ mlm_weight.T + mlm_bias

    valid_e = edge_index_pe[0] != edge_index_pe[1]
    d = jax.nn.gelu(pe @ dh_dense_w + dh_dense_b)
    d = _layer_norm(d, dh_ln_g, dh_ln_b)
    d = d @ dh_out_w + dh_out_b

    target = x[mask_nodes]
    xn = pred_node / (jnp.linalg.norm(pred_node, axis=-1, keepdims=True)
                      + 1e-12)
    yn = target / (jnp.linalg.norm(target, axis=-1, keepdims=True) + 1e-12)
    atom_loss = jnp.mean((1.0 - jnp.sum(xn * yn, axis=-1)) ** ALPHA_L)

    sentinel = 2 * N * N
    valid_f = pe_a != pe_b
    key_fwd = jnp.where(valid_f, (pe_a * N + pe_b) * 2 + mask_a, sentinel)
    key_rev = jnp.where(valid_f, (pe_b * N + pe_a) * 2 + mask_b, sentinel)
    keys_ = jnp.concatenate([key_fwd, key_rev])
    M = keys_.shape[0]
    dflat = d[:, 0]
    d2 = jnp.concatenate([dflat, dflat])
    pe2 = jnp.concatenate([PE, PE])
    idx = jnp.arange(M, dtype=jnp.int32)
    sk, ds, ps = jax.lax.sort((keys_, d2, pe2), num_keys=1)
    is_new = jnp.concatenate(
        [jnp.ones((1,), bool), sk[1:] != sk[:-1]])
    bnd = jnp.where(is_new, idx, M)
    suf = jnp.flip(jax.lax.cummin(jnp.flip(bnd)))
    s_next = jnp.concatenate([suf[1:], jnp.full((1,), M, jnp.int32)])
    e_last = s_next - 1
    dmean = jnp.mean(ds)
    pmean = jnp.mean(ps)
    Cd = jnp.cumsum(ds - dmean)
    Cp = jnp.cumsum(ps - pmean)
    Ctbl = jnp.stack([Cd, Cp, Cd, Cp], axis=-1)
    at_end = Ctbl[e_last]
    sum_d = at_end[:, 0] - (Cd - (ds - dmean))
    sum_p = at_end[:, 1] - (Cp - (ps - pmean))
    cntf = (e_last - idx + 1).astype(jnp.float32)
    rec = sum_d / cntf + dmean
    tgt = sum_p / cntf + pmean
    sel_w = (is_new & (sk != sentinel) & ((sk & 1) == 1)).astype(jnp.float32)
    dd = rec - tgt
    ad = jnp.abs(dd)
    row_loss = jnp.where(ad < 1.0, 0.5 * dd * dd, ad - 0.5)
    pe_loss = jnp.sum(row_loss * sel_w) / jnp.sum(sel_w)
    return atom_loss + pe_loss


# speedup vs baseline: 3.9895x; 1.0352x over previous
import functools

import jax
import jax.numpy as jnp
import numpy as np
from jax.experimental import pallas as pl

N = 10000
E = 320000
D = 128
EMB = 128
HEADS = 32
NUM_ATOM = 128
MASK_RATIO = 0.3
REPLACE_RATIO = 0.1
NOISE_VAL = 0.1
ALPHA_L = 2.0

NUM_MASK = int(MASK_RATIO * N)
NUM_NOISE = int(REPLACE_RATIO * NUM_MASK)
NUM_TOKEN = int((1.0 - REPLACE_RATIO) * NUM_MASK)


def _mask_constants():
    key = jax.random.key(42)
    k1, k2, k3, k4 = jax.random.split(key, 4)
    perm = jax.random.permutation(k1, N)
    mask_nodes = perm[:NUM_MASK].astype(jnp.int32)
    perm_mask = jax.random.permutation(k2, NUM_MASK)
    token_nodes = mask_nodes[perm_mask[:NUM_TOKEN]]
    noise_nodes = mask_nodes[perm_mask[-NUM_NOISE:]]
    noise_chosen = jax.random.permutation(k3, N)[:NUM_NOISE].astype(jnp.int32)
    pos_noise = NOISE_VAL * jax.random.normal(k4, (NUM_MASK, 3), jnp.float32)

    token_flag = jnp.zeros((N,), jnp.bool_).at[token_nodes].set(True)
    src_idx = jnp.arange(N, dtype=jnp.int32).at[noise_nodes].set(noise_chosen)
    u_add = jnp.zeros((N, 3), jnp.float32).at[mask_nodes].set(pos_noise)
    mask_set = jnp.zeros((N,), jnp.bool_).at[mask_nodes].set(True)
    return token_flag, src_idx, u_add, mask_nodes, mask_set


try:
    _MASK_CONSTS = tuple(
        jnp.asarray(np.asarray(c)) for c in _mask_constants())
except Exception:
    _MASK_CONSTS = None


def _layer_norm(x, g, b, eps=1e-5):
    m = jnp.mean(x, axis=-1, keepdims=True)
    v = jnp.var(x, axis=-1, keepdims=True)
    return (x - m) / jnp.sqrt(v + eps) * g + b


def _enc_body(outx_ref, agg_ref, ws_ref, wa_ref, h_ref):
    acc = jnp.dot(outx_ref[...], ws_ref[...],
                  preferred_element_type=jnp.float32)
    acc += jnp.dot(agg_ref[...], wa_ref[...],
                   preferred_element_type=jnp.float32)
    h_ref[...] = jax.nn.gelu(acc)


def _encoder_h(out_x, agg, W_self, W_agg):
    blk = 1000
    grid = N // blk
    return pl.pallas_call(
        _enc_body,
        grid=(grid,),
        in_specs=[
            pl.BlockSpec((blk, D), lambda i: (i, 0)),
            pl.BlockSpec((blk, D), lambda i: (i, 0)),
            pl.BlockSpec((D, EMB), lambda i: (0, 0)),
            pl.BlockSpec((D, EMB), lambda i: (0, 0)),
        ],
        out_specs=pl.BlockSpec((blk, EMB), lambda i: (i, 0)),
        out_shape=jax.ShapeDtypeStruct((N, EMB), jnp.float32),
    )(out_x, agg, W_self, W_agg)


def kernel(x, edge_index, u, PE, edge_index_pe, enc_mask_token, W_self,
           W_agg, W_pe, mlm_dense_w, mlm_dense_b, mlm_ln_g, mlm_ln_b,
           mlm_weight, mlm_bias, dh_dense_w, dh_dense_b, dh_ln_g, dh_ln_b,
           dh_out_w, dh_out_b):
    consts = _MASK_CONSTS if _MASK_CONSTS is not None else _mask_constants()
    token_flag, src_idx, u_add, mask_nodes, mask_set = consts

    out_x = jnp.where(token_flag[:, None], enc_mask_token[0][None, :],
                      x[src_idx])
    u_masked = u + u_add

    pe_a, pe_b = edge_index_pe[0], edge_index_pe[1]
    maskf = mask_set.astype(jnp.float32)
    U16 = jnp.concatenate(
        [u_masked, maskf[:, None], jnp.zeros((N, 12), jnp.float32)], axis=1)
    Ua = U16[pe_a]
    Ub = U16[pe_b]
    du = Ua[:, :3] - Ub[:, :3]
    PE_noise = jnp.sqrt(jnp.sum(du * du, axis=-1) + 1e-12)
    mask_a = Ua[:, 3].astype(jnp.int32)
    mask_b = Ub[:, 3].astype(jnp.int32)

    src, dst = edge_index[0], edge_index[1]
    agg = jax.ops.segment_sum(out_x[src], dst, num_segments=N)
    deg = jax.ops.segment_sum(jnp.ones((E,), jnp.float32), dst,
                              num_segments=N)
    agg = agg / jnp.maximum(deg, 1.0)[:, None]

    h = _encoder_h(out_x, agg, W_self, W_agg)

    hW = h @ W_pe
    pe = jnp.tanh(hW[edge_index_pe[0]] - hW[edge_index_pe[1]]
                  + (PE - PE_noise)[:, None])

    feats = h[mask_nodes]
    z = jax.nn.gelu(feats @ mlm_dense_w + mlm_dense_b)
    z = _layer_norm(z, mlm_ln_g, mlm_ln_b)
    pred_node = z @ mlm_weight.T + mlm_bias

    valid_e = edge_index_pe[0] != edge_index_pe[1]
    d = jax.nn.gelu(pe @ dh_dense_w + dh_dense_b)
    d = _layer_norm(d, dh_ln_g, dh_ln_b)
    d = d @ dh_out_w + dh_out_b

    target = x[mask_nodes]
    xn = pred_node / (jnp.linalg.norm(pred_node, axis=-1, keepdims=True)
                      + 1e-12)
    yn = target / (jnp.linalg.norm(target, axis=-1, keepdims=True) + 1e-12)
    atom_loss = jnp.mean((1.0 - jnp.sum(xn * yn, axis=-1)) ** ALPHA_L)

    sentinel = 2 * N * N
    valid_f = pe_a != pe_b
    key_fwd = jnp.where(valid_f, (pe_a * N + pe_b) * 2 + mask_a, sentinel)
    key_rev = jnp.where(valid_f, (pe_b * N + pe_a) * 2 + mask_b, sentinel)
    keys_ = jnp.concatenate([key_fwd, key_rev])
    M = keys_.shape[0]
    dflat = d[:, 0]
    d2 = jnp.concatenate([dflat, dflat])
    pe2 = jnp.concatenate([PE, PE])
    idx = jnp.arange(M, dtype=jnp.int32)
    sk, ds, ps = jax.lax.sort((keys_, d2, pe2), num_keys=1)
    is_new = jnp.concatenate(
        [jnp.ones((1,), bool), sk[1:] != sk[:-1]])
    bnd = jnp.where(is_new, idx, M)
    suf = jnp.flip(jax.lax.cummin(jnp.flip(bnd)))
    s_next = jnp.concatenate([suf[1:], jnp.full((1,), M, jnp.int32)])
    e_last = s_next - 1
    dmean = jnp.mean(ds)
    pmean = jnp.mean(ps)
    Cd = jnp.cumsum(ds - dmean)
    Cp = jnp.cumsum(ps - pmean)
    Ctbl = jnp.stack([Cd, Cp, Cd, Cp], axis=-1)
    at_end = Ctbl[e_last]
    sum_d = at_end[:, 0] - (Cd - (ds - dmean))
    sum_p = at_end[:, 1] - (Cp - (ps - pmean))
    cntf = (e_last - idx + 1).astype(jnp.float32)
    rec = sum_d / cntf + dmean
    tgt = sum_p / cntf + pmean
    sel_w = (is_new & (sk != sentinel) & ((sk & 1) == 1)).astype(jnp.float32)
    dd = rec - tgt
    ad = jnp.abs(dd)
    row_loss = jnp.where(ad < 1.0, 0.5 * dd * dd, ad - 0.5)
    pe_loss = jnp.sum(row_loss * sel_w) / jnp.sum(sel_w)
    return atom_loss + pe_loss


# speedup vs baseline: 4.0472x; 1.0145x over previous
import functools

import jax
import jax.numpy as jnp
import numpy as np
from jax.experimental import pallas as pl

N = 10000
E = 320000
D = 128
EMB = 128
HEADS = 32
NUM_ATOM = 128
MASK_RATIO = 0.3
REPLACE_RATIO = 0.1
NOISE_VAL = 0.1
ALPHA_L = 2.0

NUM_MASK = int(MASK_RATIO * N)
NUM_NOISE = int(REPLACE_RATIO * NUM_MASK)
NUM_TOKEN = int((1.0 - REPLACE_RATIO) * NUM_MASK)


def _mask_constants():
    key = jax.random.key(42)
    k1, k2, k3, k4 = jax.random.split(key, 4)
    perm = jax.random.permutation(k1, N)
    mask_nodes = perm[:NUM_MASK].astype(jnp.int32)
    perm_mask = jax.random.permutation(k2, NUM_MASK)
    token_nodes = mask_nodes[perm_mask[:NUM_TOKEN]]
    noise_nodes = mask_nodes[perm_mask[-NUM_NOISE:]]
    noise_chosen = jax.random.permutation(k3, N)[:NUM_NOISE].astype(jnp.int32)
    pos_noise = NOISE_VAL * jax.random.normal(k4, (NUM_MASK, 3), jnp.float32)

    token_flag = jnp.zeros((N,), jnp.bool_).at[token_nodes].set(True)
    src_idx = jnp.arange(N, dtype=jnp.int32).at[noise_nodes].set(noise_chosen)
    u_add = jnp.zeros((N, 3), jnp.float32).at[mask_nodes].set(pos_noise)
    mask_set = jnp.zeros((N,), jnp.bool_).at[mask_nodes].set(True)
    return token_flag, src_idx, u_add, mask_nodes, mask_set


try:
    _MASK_CONSTS = tuple(
        jnp.asarray(np.asarray(c)) for c in _mask_constants())
except Exception:
    _MASK_CONSTS = None


def _row_gather(table, indices):
    table, indices = jax.lax.optimization_barrier((table, indices))
    return jax.lax.optimization_barrier(table[indices])


def _layer_norm(x, g, b, eps=1e-5):
    m = jnp.mean(x, axis=-1, keepdims=True)
    v = jnp.var(x, axis=-1, keepdims=True)
    return (x - m) / jnp.sqrt(v + eps) * g + b


def _enc_body(outx_ref, agg_ref, ws_ref, wa_ref, h_ref):
    acc = jnp.dot(outx_ref[...], ws_ref[...],
                  preferred_element_type=jnp.float32)
    acc += jnp.dot(agg_ref[...], wa_ref[...],
                   preferred_element_type=jnp.float32)
    h_ref[...] = jax.nn.gelu(acc)


def _encoder_h(out_x, agg, W_self, W_agg):
    blk = 1000
    grid = N // blk
    return pl.pallas_call(
        _enc_body,
        grid=(grid,),
        in_specs=[
            pl.BlockSpec((blk, D), lambda i: (i, 0)),
            pl.BlockSpec((blk, D), lambda i: (i, 0)),
            pl.BlockSpec((D, EMB), lambda i: (0, 0)),
            pl.BlockSpec((D, EMB), lambda i: (0, 0)),
        ],
        out_specs=pl.BlockSpec((blk, EMB), lambda i: (i, 0)),
        out_shape=jax.ShapeDtypeStruct((N, EMB), jnp.float32),
    )(out_x, agg, W_self, W_agg)


def kernel(x, edge_index, u, PE, edge_index_pe, enc_mask_token, W_self,
           W_agg, W_pe, mlm_dense_w, mlm_dense_b, mlm_ln_g, mlm_ln_b,
           mlm_weight, mlm_bias, dh_dense_w, dh_dense_b, dh_ln_g, dh_ln_b,
           dh_out_w, dh_out_b):
    consts = _MASK_CONSTS if _MASK_CONSTS is not None else _mask_constants()
    token_flag, src_idx, u_add, mask_nodes, mask_set = consts

    out_x = jnp.where(token_flag[:, None], enc_mask_token[0][None, :],
                      _row_gather(x, src_idx))
    u_masked = u + u_add

    pe_a, pe_b = edge_index_pe[0], edge_index_pe[1]
    maskf = mask_set.astype(jnp.float32)
    U16 = jnp.concatenate(
        [u_masked, maskf[:, None], jnp.zeros((N, 12), jnp.float32)], axis=1)
    Ua = _row_gather(U16, pe_a)
    Ub = _row_gather(U16, pe_b)
    du = Ua[:, :3] - Ub[:, :3]
    PE_noise = jnp.sqrt(jnp.sum(du * du, axis=-1) + 1e-12)
    mask_a = Ua[:, 3].astype(jnp.int32)
    mask_b = Ub[:, 3].astype(jnp.int32)

    src, dst = edge_index[0], edge_index[1]
    agg = jax.ops.segment_sum(out_x[src], dst, num_segments=N)
    deg = jax.ops.segment_sum(jnp.ones((E,), jnp.float32), dst,
                              num_segments=N)
    agg = agg / jnp.maximum(deg, 1.0)[:, None]

    h = _encoder_h(out_x, agg, W_self, W_agg)

    hW = h @ W_pe
    pe = jnp.tanh(hW[edge_index_pe[0]] - hW[edge_index_pe[1]]
                  + (PE - PE_noise)[:, None])

    feats = _row_gather(h, mask_nodes)
    z = jax.nn.gelu(feats @ mlm_dense_w + mlm_dense_b)
    z = _layer_norm(z, mlm_ln_g, mlm_ln_b)
    pred_node = z @ mlm_weight.T + mlm_bias

    valid_e = edge_index_pe[0] != edge_index_pe[1]
    d = jax.nn.gelu(pe @ dh_dense_w + dh_dense_b)
    d = _layer_norm(d, dh_ln_g, dh_ln_b)
    d = d @ dh_out_w + dh_out_b

    target = _row_gather(x, mask_nodes)
    xn = pred_node / (jnp.linalg.norm(pred_node, axis=-1, keepdims=True)
                      + 1e-12)
    yn = target / (jnp.linalg.norm(target, axis=-1, keepdims=True) + 1e-12)
    atom_loss = jnp.mean((1.0 - jnp.sum(xn * yn, axis=-1)) ** ALPHA_L)

    sentinel = 2 * N * N
    valid_f = pe_a != pe_b
    key_fwd = jnp.where(valid_f, (pe_a * N + pe_b) * 2 + mask_a, sentinel)
    key_rev = jnp.where(valid_f, (pe_b * N + pe_a) * 2 + mask_b, sentinel)
    keys_ = jnp.concatenate([key_fwd, key_rev])
    M = keys_.shape[0]
    dflat = d[:, 0]
    d2 = jnp.concatenate([dflat, dflat])
    pe2 = jnp.concatenate([PE, PE])
    idx = jnp.arange(M, dtype=jnp.int32)
    sk, ds, ps = jax.lax.sort((keys_, d2, pe2), num_keys=1)
    is_new = jnp.concatenate(
        [jnp.ones((1,), bool), sk[1:] != sk[:-1]])
    bnd = jnp.where(is_new, idx, M)
    suf = jnp.flip(jax.lax.cummin(jnp.flip(bnd)))
    s_next = jnp.concatenate([suf[1:], jnp.full((1,), M, jnp.int32)])
    e_last = s_next - 1
    dmean = jnp.mean(ds)
    pmean = jnp.mean(ps)
    Cd = jnp.cumsum(ds - dmean)
    Cp = jnp.cumsum(ps - pmean)
    Ctbl = jnp.stack([Cd, Cp, Cd, Cp], axis=-1)
    at_end = _row_gather(Ctbl, e_last)
    sum_d = at_end[:, 0] - (Cd - (ds - dmean))
    sum_p = at_end[:, 1] - (Cp - (ps - pmean))
    cntf = (e_last - idx + 1).astype(jnp.float32)
    rec = sum_d / cntf + dmean
    tgt = sum_p / cntf + pmean
    sel_w = (is_new & (sk != sentinel) & ((sk & 1) == 1)).astype(jnp.float32)
    dd = rec - tgt
    ad = jnp.abs(dd)
    row_loss = jnp.where(ad < 1.0, 0.5 * dd * dd, ad - 0.5)
    pe_loss = jnp.sum(row_loss * sel_w) / jnp.sum(sel_w)
    return atom_loss + pe_loss
